# baseline (speedup 1.0000x reference)
import jax
import jax.numpy as jnp
from jax import lax
from jax.experimental import pallas as pl
from jax.experimental.pallas import tpu as pltpu
from jax.experimental.pallas import tpu_sc as plsc

_RATE = 0.04
_NROWS = 32
_D = 512
_GROUP = 8
_NSUB = 16
_LANES = 16
_CBLK = 128
_NCB = _D // _CBLK
_C1 = _RATE * float(_GROUP) / float(_GROUP - 1)
_C2 = _RATE / float(_GROUP - 1)


def _regularizer_body(in_hbm, out_hbm, block_v, src_v, gather_v, shared_v,
                      sem):
    s = lax.axis_index("s")
    zero = jnp.zeros((_LANES,), jnp.float32)

    g = s // _NCB
    b = s - g * _NCB
    r0 = pl.multiple_of(g * _GROUP, _GROUP)
    c0 = pl.multiple_of(b * _CBLK, _CBLK)
    cp = pltpu.async_copy(in_hbm.at[pl.ds(r0, _GROUP), pl.ds(c0, _CBLK)],
                          block_v, sem)

    @pl.when(s == _NSUB - 1)
    def _init_accumulator():
        src_v[...] = zero
        pltpu.sync_copy(src_v, shared_v)

    cp.wait()
    sq_parts = []
    gsq_parts = []
    for k in range(_CBLK // _LANES):
        sq_k = zero
        ssum = zero
        for i in range(_GROUP):
            v = block_v[i, pl.ds(k * _LANES, _LANES)]
            sq_k = sq_k + v * v
            ssum = ssum + v
        sq_parts.append(sq_k)
        gsq_parts.append(ssum * ssum)
    while len(sq_parts) > 1:
        sq_parts = [a + b for a, b in zip(sq_parts[::2], sq_parts[1::2])]
    while len(gsq_parts) > 1:
        gsq_parts = [a + b for a, b in zip(gsq_parts[::2], gsq_parts[1::2])]
    part = _C1 * sq_parts[0] - _C2 * gsq_parts[0]
    total = part[0]
    for i in range(1, _LANES):
        total = total + part[i]
    src_v[...] = jnp.full((_LANES,), total, jnp.float32)

    plsc.subcore_barrier()

    pltpu.sync_copy(src_v, shared_v.at[lax.iota(jnp.int32, _LANES)],
                    add=True)

    plsc.subcore_barrier()

    @pl.when(s == 0)
    def _write_out():
        pltpu.sync_copy(shared_v, gather_v)
        pltpu.sync_copy(gather_v, out_hbm)


@jax.jit
def _regularizer(inputs):
    kern = pl.kernel(
        _regularizer_body,
        out_type=jax.ShapeDtypeStruct((_LANES,), jnp.float32),
        mesh=plsc.VectorSubcoreMesh(core_axis_name="c", subcore_axis_name="s",
                                    num_cores=1),
        scratch_types=[
            pltpu.VMEM((_GROUP, _CBLK), jnp.float32),
            pltpu.VMEM((_LANES,), jnp.float32),
            pltpu.VMEM((_LANES,), jnp.float32),
            pltpu.VMEM_SHARED((_LANES,), jnp.float32),
            pltpu.SemaphoreType.DMA,
        ],
    )
    return kern(inputs)[0]


def kernel(inputs):
    return _regularizer(inputs)


# speedup vs baseline: 1.0039x; 1.0039x over previous
import jax
import jax.numpy as jnp
from jax import lax
from jax.experimental import pallas as pl
from jax.experimental.pallas import tpu as pltpu
from jax.experimental.pallas import tpu_sc as plsc

_RATE = 0.04
_NROWS = 32
_D = 512
_GROUP = 8
_NSUB = 16
_LANES = 16
_CBLK = 128
_NCB = _D // _CBLK
_C1 = _RATE * float(_GROUP) / float(_GROUP - 1)
_C2 = _RATE / float(_GROUP - 1)


def _regularizer_body(in_hbm, out_hbm, block_v, src_v, gather_v, shared_v,
                      sem):
    s = lax.axis_index("s")
    zero = jnp.zeros((_LANES,), jnp.float32)

    g = s // _NCB
    b = s - g * _NCB
    r0 = pl.multiple_of(g * _GROUP, _GROUP)
    c0 = pl.multiple_of(b * _CBLK, _CBLK)
    cp = pltpu.async_copy(in_hbm.at[pl.ds(r0, _GROUP), pl.ds(c0, _CBLK)],
                          block_v, sem)

    @pl.when(s == _NSUB - 1)
    def _init_accumulator():
        src_v[...] = zero
        pltpu.sync_copy(src_v, shared_v)

    cp.wait()
    sq_parts = []
    gsq_parts = []
    for k in range(_CBLK // _LANES):
        sq_k = zero
        ssum = zero
        for i in range(_GROUP):
            v = block_v[i, pl.ds(k * _LANES, _LANES)]
            sq_k = sq_k + v * v
            ssum = ssum + v
        sq_parts.append(sq_k)
        gsq_parts.append(ssum * ssum)
    while len(sq_parts) > 1:
        sq_parts = [a + b for a, b in zip(sq_parts[::2], sq_parts[1::2])]
    while len(gsq_parts) > 1:
        gsq_parts = [a + b for a, b in zip(gsq_parts[::2], gsq_parts[1::2])]
    part = _C1 * sq_parts[0] - _C2 * gsq_parts[0]
    total = part[0]
    for i in range(1, _LANES):
        total = total + part[i]
    src_v[...] = jnp.full((_LANES,), total, jnp.float32)

    plsc.subcore_barrier()

    pltpu.sync_copy(src_v, shared_v.at[lax.iota(jnp.int32, _LANES)],
                    add=True)

    plsc.subcore_barrier()

    @pl.when(s == 0)
    def _write_out():
        pltpu.sync_copy(shared_v, gather_v)
        pltpu.sync_copy(gather_v, out_hbm)


@jax.jit
def _regularizer(inputs):
    kern = pl.kernel(
        _regularizer_body,
        out_type=jax.ShapeDtypeStruct((_LANES,), jnp.float32),
        mesh=plsc.VectorSubcoreMesh(core_axis_name="c", subcore_axis_name="s",
                                    num_cores=1),
        compiler_params=pltpu.CompilerParams(skip_device_barrier=True),
        scratch_types=[
            pltpu.VMEM((_GROUP, _CBLK), jnp.float32),
            pltpu.VMEM((_LANES,), jnp.float32),
            pltpu.VMEM((_LANES,), jnp.float32),
            pltpu.VMEM_SHARED((_LANES,), jnp.float32),
            pltpu.SemaphoreType.DMA,
        ],
    )
    return kern(inputs)[0]


def kernel(inputs):
    return _regularizer(inputs)


# speedup vs baseline: 1.0090x; 1.0051x over previous
import jax
import jax.numpy as jnp
from jax import lax
from jax.experimental import pallas as pl
from jax.experimental.pallas import tpu as pltpu
from jax.experimental.pallas import tpu_sc as plsc

_RATE = 0.04
_NROWS = 32
_D = 512
_GROUP = 8
_NSUB = 16
_LANES = 16
_CBLK = 128
_NCB = _D // _CBLK
_C1 = _RATE * float(_GROUP) / float(_GROUP - 1)
_C2 = _RATE / float(_GROUP - 1)


def _regularizer_body(in_hbm, out_hbm, block_v, src_v, gather_v, shared_v,
                      sem):
    s = lax.axis_index("s")
    zero = jnp.zeros((_LANES,), jnp.float32)

    g = s // _NCB
    b = s - g * _NCB
    r0 = pl.multiple_of(g * _GROUP, _GROUP)
    c0 = pl.multiple_of(b * _CBLK, _CBLK)
    cp = pltpu.async_copy(in_hbm.at[pl.ds(r0, _GROUP), pl.ds(c0, _CBLK)],
                          block_v, sem)

    @pl.when(s == _NSUB - 1)
    def _init_accumulator():
        src_v[...] = zero
        pltpu.sync_copy(src_v, shared_v)

    cp.wait()
    sq_parts = []
    gsq_parts = []
    for k in range(_CBLK // _LANES):
        sq_k = zero
        ssum = zero
        for i in range(_GROUP):
            v = block_v[i, pl.ds(k * _LANES, _LANES)]
            sq_k = sq_k + v * v
            ssum = ssum + v
        sq_parts.append(sq_k)
        gsq_parts.append(ssum * ssum)
    while len(sq_parts) > 1:
        sq_parts = [a + b for a, b in zip(sq_parts[::2], sq_parts[1::2])]
    while len(gsq_parts) > 1:
        gsq_parts = [a + b for a, b in zip(gsq_parts[::2], gsq_parts[1::2])]
    part = _C1 * sq_parts[0] - _C2 * gsq_parts[0]
    total = part[0]
    for i in range(1, _LANES):
        total = total + part[i]
    src_v[...] = jnp.full((_LANES,), total, jnp.float32)

    plsc.subcore_barrier()

    pltpu.sync_copy(src_v, shared_v.at[lax.iota(jnp.int32, _LANES)],
                    add=True)

    plsc.subcore_barrier()

    @pl.when(s == 0)
    def _write_out():
        pltpu.sync_copy(shared_v, gather_v)
        pltpu.sync_copy(gather_v, out_hbm)


@jax.jit
def _regularizer(inputs):
    kern = pl.kernel(
        _regularizer_body,
        out_type=jax.ShapeDtypeStruct((_LANES,), jnp.float32),
        mesh=plsc.VectorSubcoreMesh(core_axis_name="c", subcore_axis_name="s",
                                    num_cores=1),
        compiler_params=pltpu.CompilerParams(use_tc_tiling_on_sc=False),
        scratch_types=[
            pltpu.VMEM((_GROUP, _CBLK), jnp.float32),
            pltpu.VMEM((_LANES,), jnp.float32),
            pltpu.VMEM((_LANES,), jnp.float32),
            pltpu.VMEM_SHARED((_LANES,), jnp.float32),
            pltpu.SemaphoreType.DMA,
        ],
    )
    return kern(inputs)[0]


def kernel(inputs):
    return _regularizer(inputs)
